# baseline (speedup 1.0000x reference)
import jax
import jax.numpy as jnp
from jax import lax
from jax.experimental import pallas as pl
from jax.experimental.pallas import tpu as pltpu
from jax.experimental.pallas import tpu_sc as plsc

N_FEATURES = 26
VOCAB = 100001
EMBED = 32
BATCH = 4096
SEQ = 20

N = BATCH * SEQ
NC, NS, L = 2, 16, 16
NW = NC * NS
PER_W = N // NW
B = 16
ROWS = B * N_FEATURES
N_CHUNKS = PER_W // B
GROUPS = (N_FEATURES * VOCAB + 2) // 4
WR = ROWS // 4


def _embed_body(idx_hbm, offs_hbm, table_hbm, out_hbm,
                offs_v, sub0, sub1, idx0, idx1, g0, g1,
                isem0, isem1, gsem0, gsem1, wsem0, wsem1):
  wid = lax.axis_index("s") * NC + lax.axis_index("c")
  row0_w = wid * (PER_W * N_FEATURES)
  wrow0_w = wid * (PER_W * N_FEATURES // 4)

  pltpu.sync_copy(offs_hbm, offs_v)

  def idx_load(g, idx_v, isem):
    pltpu.async_copy(idx_hbm.at[pl.ds(row0_w + g * ROWS, ROWS)], idx_v, isem)

  def idx_wait(g, idx_v, isem):
    pltpu.make_async_copy(
        idx_hbm.at[pl.ds(row0_w + g * ROWS, ROWS)], idx_v, isem).wait()

  def offset_add(idx_v, sub_v):
    def vec_body(j, carry):
      sl = pl.ds(j * L, L)
      t = idx_v[sl] + offs_v[sl]
      sub_v[sl] = lax.bitwise_and(t, 3)
      idx_v[sl] = lax.shift_right_logical(t, 2)
      return carry
    lax.fori_loop(0, ROWS // L, vec_body, 0)

  def extract(g_v, sub_v):
    iota = lax.iota(jnp.int32, L)

    def blk(t, carry):
      q = t * L
      sub16 = sub_v[pl.ds(q, L)]
      row_r = q + iota
      col_base = sub16 * EMBED
      wflat = (q + iota) * EMBED

      def e_body(e, carry2):
        v = plsc.load_gather(g_v, [row_r, col_base + e])
        wf = wflat + e
        plsc.store_scatter(
            g_v, [lax.shift_right_logical(wf, 7), lax.bitwise_and(wf, 127)], v)
        return carry2

      lax.fori_loop(0, EMBED, e_body, 0)
      return carry

    lax.fori_loop(0, ROWS // L, blk, 0)

  KS = 13
  SUB = ROWS // KS

  def gather_fire(idx_v, g_v, gsem):
    pltpu.async_copy(table_hbm.at[idx_v], g_v, gsem)

  def gather_wait(idx_v, g_v, gsem):
    pltpu.make_async_copy(table_hbm.at[idx_v], g_v, gsem).wait()

  def write_fire(g, g_v, wsem):
    pltpu.async_copy(g_v.at[pl.ds(0, WR)],
                     out_hbm.at[pl.ds(wrow0_w + g * WR, WR)], wsem)

  def write_wait(g, g_v, wsem):
    pltpu.make_async_copy(g_v.at[pl.ds(0, WR)],
                          out_hbm.at[pl.ds(wrow0_w + g * WR, WR)], wsem).wait()

  bufs = ((idx0, sub0, g0, isem0, gsem0, wsem0),
          (idx1, sub1, g1, isem1, gsem1, wsem1))

  idx_load(0, idx0, isem0)
  idx_load(1, idx1, isem1)

  def chunk_body(h, carry):
    for par in (0, 1):
      idx_v, sub_v, g_v, isem, gsem, wsem = bufs[par]
      g = h * 2 + par

      @pl.when(g >= 2)
      def _():
        write_wait(g - 2, g_v, wsem)

      idx_wait(g, idx_v, isem)
      offset_add(idx_v, sub_v)
      gather_fire(idx_v, g_v, gsem)
      gather_wait(idx_v, g_v, gsem)

      @pl.when(g + 2 < N_CHUNKS)
      def _():
        idx_load(g + 2, idx_v, isem)

      extract(g_v, sub_v)
      write_fire(g, g_v, wsem)
    return carry

  lax.fori_loop(0, N_CHUNKS // 2, chunk_body, 0)

  write_wait(N_CHUNKS - 2, g0, wsem0)
  write_wait(N_CHUNKS - 1, g1, wsem1)


@jax.jit
def kernel(features, tables):
  idx = features.reshape(N_FEATURES, N).T.reshape(N * N_FEATURES)
  offs = jnp.tile(jnp.arange(N_FEATURES, dtype=jnp.int32) * VOCAB, B)
  flat = tables.reshape(N_FEATURES * VOCAB * EMBED)
  flat = jnp.concatenate([flat, jnp.zeros(64, jnp.float32)])
  table = flat.reshape(GROUPS, 128)
  mesh = plsc.VectorSubcoreMesh(core_axis_name="c", subcore_axis_name="s")
  out = pl.kernel(
      _embed_body,
      out_type=jax.ShapeDtypeStruct((N * N_FEATURES // 4, 128), jnp.float32),
      mesh=mesh,
      scratch_types=[
          pltpu.VMEM((ROWS,), jnp.int32),
          pltpu.VMEM((ROWS,), jnp.int32),
          pltpu.VMEM((ROWS,), jnp.int32),
          pltpu.VMEM((ROWS,), jnp.int32),
          pltpu.VMEM((ROWS,), jnp.int32),
          pltpu.VMEM((ROWS, 128), jnp.float32),
          pltpu.VMEM((ROWS, 128), jnp.float32),
          pltpu.SemaphoreType.DMA,
          pltpu.SemaphoreType.DMA,
          pltpu.SemaphoreType.DMA,
          pltpu.SemaphoreType.DMA,
          pltpu.SemaphoreType.DMA,
          pltpu.SemaphoreType.DMA,
      ],
      compiler_params=pltpu.CompilerParams(needs_layout_passes=False),
  )(idx, offs, table)
  return out.reshape(BATCH, SEQ, N_FEATURES * EMBED)


# speedup vs baseline: 1.0432x; 1.0432x over previous
import jax
import jax.numpy as jnp
from jax import lax
from jax.experimental import pallas as pl
from jax.experimental.pallas import tpu as pltpu
from jax.experimental.pallas import tpu_sc as plsc

N_FEATURES = 26
VOCAB = 100001
EMBED = 32
BATCH = 4096
SEQ = 20

N = BATCH * SEQ
NC, NS, L = 2, 16, 16
NW = NC * NS
PER_W = N // NW
B = 16
ROWS = B * N_FEATURES
N_CHUNKS = PER_W // B
GROUPS = (N_FEATURES * VOCAB + 2) // 4
WR = ROWS // 4


def _embed_body(idx_hbm, offs_hbm, table_hbm, out_hbm,
                offs_v, sub0, sub1, idx0, idx1, g0, g1,
                isem0, isem1, gsem0, gsem1, wsem0, wsem1):
  wid = lax.axis_index("s") * NC + lax.axis_index("c")
  row0_w = wid * (PER_W * N_FEATURES)
  wrow0_w = wid * (PER_W * N_FEATURES // 4)

  pltpu.sync_copy(offs_hbm, offs_v)

  def idx_load(g, idx_v, isem):
    pltpu.async_copy(idx_hbm.at[pl.ds(row0_w + g * ROWS, ROWS)], idx_v, isem)

  def idx_wait(g, idx_v, isem):
    pltpu.make_async_copy(
        idx_hbm.at[pl.ds(row0_w + g * ROWS, ROWS)], idx_v, isem).wait()

  def offset_add(idx_v, sub_v):
    def vec_body(j, carry):
      sl = pl.ds(j * L, L)
      t = idx_v[sl] + offs_v[sl]
      sub_v[sl] = lax.bitwise_and(t, 3)
      idx_v[sl] = lax.shift_right_logical(t, 2)
      return carry
    lax.fori_loop(0, ROWS // L, vec_body, 0)

  def extract(g_v, sub_v):
    iota = lax.iota(jnp.int32, L)

    def blk(t, carry):
      q = t * L
      sub16 = sub_v[pl.ds(q, L)]
      row_r = q + iota
      col_base = sub16 * EMBED
      wflat = (q + iota) * EMBED

      def e_body(e, carry2):
        v = plsc.load_gather(g_v, [row_r, col_base + e])
        wf = wflat + e
        plsc.store_scatter(
            g_v, [lax.shift_right_logical(wf, 7), lax.bitwise_and(wf, 127)], v)
        return carry2

      lax.fori_loop(0, EMBED, e_body, 0)
      return carry

    lax.fori_loop(0, ROWS // L, blk, 0)

  KS = 13
  SUB = ROWS // KS

  def gather_fire(idx_v, g_v, gsem):
    pltpu.async_copy(table_hbm.at[idx_v], g_v, gsem)

  def gather_wait(idx_v, g_v, gsem):
    pltpu.make_async_copy(table_hbm.at[idx_v], g_v, gsem).wait()

  def write_fire(g, g_v, wsem):
    pltpu.async_copy(g_v.at[pl.ds(0, WR)],
                     out_hbm.at[pl.ds(wrow0_w + g * WR, WR)], wsem)

  def write_wait(g, g_v, wsem):
    pltpu.make_async_copy(g_v.at[pl.ds(0, WR)],
                          out_hbm.at[pl.ds(wrow0_w + g * WR, WR)], wsem).wait()

  bufs = ((idx0, sub0, g0, isem0, gsem0, wsem0),
          (idx1, sub1, g1, isem1, gsem1, wsem1))

  idx_load(0, idx0, isem0)
  idx_load(1, idx1, isem1)

  def chunk_body(h, carry):
    for par in (0, 1):
      idx_v, sub_v, g_v, isem, gsem, wsem = bufs[par]
      idx_p, sub_p, g_p, isem_p, gsem_p, wsem_p = bufs[1 - par]
      g = h * 2 + par

      @pl.when(g >= 2)
      def _():
        write_wait(g - 2, g_v, wsem)

      idx_wait(g, idx_v, isem)
      offset_add(idx_v, sub_v)
      gather_fire(idx_v, g_v, gsem)

      @pl.when(g >= 1)
      def _():
        gather_wait(idx_p, g_p, gsem_p)

        @pl.when(g + 1 < N_CHUNKS)
        def _():
          idx_load(g + 1, idx_p, isem_p)

        extract(g_p, sub_p)
        write_fire(g - 1, g_p, wsem_p)
    return carry

  lax.fori_loop(0, N_CHUNKS // 2, chunk_body, 0)

  lastp = (N_CHUNKS - 1) & 1
  idx_l, sub_l, g_l, isem_l, gsem_l, wsem_l = bufs[lastp]
  gather_wait(idx_l, g_l, gsem_l)
  extract(g_l, sub_l)
  write_fire(N_CHUNKS - 1, g_l, wsem_l)
  write_wait(N_CHUNKS - 2, g0 if lastp else g1,
             wsem0 if lastp else wsem1)
  write_wait(N_CHUNKS - 1, g_l, wsem_l)


@jax.jit
def kernel(features, tables):
  idx = features.reshape(N_FEATURES, N).T.reshape(N * N_FEATURES)
  offs = jnp.tile(jnp.arange(N_FEATURES, dtype=jnp.int32) * VOCAB, B)
  flat = tables.reshape(N_FEATURES * VOCAB * EMBED)
  flat = jnp.concatenate([flat, jnp.zeros(64, jnp.float32)])
  table = flat.reshape(GROUPS, 128)
  mesh = plsc.VectorSubcoreMesh(core_axis_name="c", subcore_axis_name="s")
  out = pl.kernel(
      _embed_body,
      out_type=jax.ShapeDtypeStruct((N * N_FEATURES // 4, 128), jnp.float32),
      mesh=mesh,
      scratch_types=[
          pltpu.VMEM((ROWS,), jnp.int32),
          pltpu.VMEM((ROWS,), jnp.int32),
          pltpu.VMEM((ROWS,), jnp.int32),
          pltpu.VMEM((ROWS,), jnp.int32),
          pltpu.VMEM((ROWS,), jnp.int32),
          pltpu.VMEM((ROWS, 128), jnp.float32),
          pltpu.VMEM((ROWS, 128), jnp.float32),
          pltpu.SemaphoreType.DMA,
          pltpu.SemaphoreType.DMA,
          pltpu.SemaphoreType.DMA,
          pltpu.SemaphoreType.DMA,
          pltpu.SemaphoreType.DMA,
          pltpu.SemaphoreType.DMA,
      ],
      compiler_params=pltpu.CompilerParams(needs_layout_passes=False),
  )(idx, offs, table)
  return out.reshape(BATCH, SEQ, N_FEATURES * EMBED)


# speedup vs baseline: 1.0435x; 1.0002x over previous
import jax
import jax.numpy as jnp
from jax import lax
from jax.experimental import pallas as pl
from jax.experimental.pallas import tpu as pltpu
from jax.experimental.pallas import tpu_sc as plsc

N_FEATURES = 26
VOCAB = 100001
EMBED = 32
BATCH = 4096
SEQ = 20

N = BATCH * SEQ
NC, NS, L = 2, 16, 16
NW = NC * NS
PER_W = N // NW
B = 16
ROWS = B * N_FEATURES
N_CHUNKS = PER_W // B
GROUPS = (N_FEATURES * VOCAB + 2) // 4
WR = ROWS // 4


def _embed_body(idx_hbm, offs_hbm, table_hbm, out_hbm,
                offs_v, sub0, sub1, idx0, idx1, g0, g1,
                isem0, isem1, gsem0, gsem1, wsem0, wsem1):
  wid = lax.axis_index("s") * NC + lax.axis_index("c")
  row0_w = wid * (PER_W * N_FEATURES)
  wrow0_w = wid * (PER_W * N_FEATURES // 4)

  pltpu.sync_copy(offs_hbm, offs_v)

  def idx_load(g, idx_v, isem):
    pltpu.async_copy(idx_hbm.at[pl.ds(row0_w + g * ROWS, ROWS)], idx_v, isem)

  def idx_wait(g, idx_v, isem):
    pltpu.make_async_copy(
        idx_hbm.at[pl.ds(row0_w + g * ROWS, ROWS)], idx_v, isem).wait()

  def offset_add(idx_v, sub_v):
    def vec_body(j, carry):
      sl = pl.ds(j * L, L)
      t = idx_v[sl] + offs_v[sl]
      sub_v[sl] = lax.bitwise_and(t, 3)
      idx_v[sl] = lax.shift_right_logical(t, 2)
      return carry
    lax.fori_loop(0, ROWS // L, vec_body, 0, unroll=8)

  def extract(g_v, sub_v):
    iota = lax.iota(jnp.int32, L)

    def blk(t, carry):
      q = t * L
      sub16 = sub_v[pl.ds(q, L)]
      row_r = q + iota
      col_base = sub16 * EMBED
      wflat = (q + iota) * EMBED

      def e_body(e, carry2):
        v = plsc.load_gather(g_v, [row_r, col_base + e])
        wf = wflat + e
        plsc.store_scatter(
            g_v, [lax.shift_right_logical(wf, 7), lax.bitwise_and(wf, 127)], v)
        return carry2

      lax.fori_loop(0, EMBED, e_body, 0, unroll=True)
      return carry

    lax.fori_loop(0, ROWS // L, blk, 0)

  KS = 13
  SUB = ROWS // KS

  def gather_fire(idx_v, g_v, gsem):
    pltpu.async_copy(table_hbm.at[idx_v], g_v, gsem)

  def gather_wait(idx_v, g_v, gsem):
    pltpu.make_async_copy(table_hbm.at[idx_v], g_v, gsem).wait()

  def write_fire(g, g_v, wsem):
    pltpu.async_copy(g_v.at[pl.ds(0, WR)],
                     out_hbm.at[pl.ds(wrow0_w + g * WR, WR)], wsem)

  def write_wait(g, g_v, wsem):
    pltpu.make_async_copy(g_v.at[pl.ds(0, WR)],
                          out_hbm.at[pl.ds(wrow0_w + g * WR, WR)], wsem).wait()

  bufs = ((idx0, sub0, g0, isem0, gsem0, wsem0),
          (idx1, sub1, g1, isem1, gsem1, wsem1))

  idx_load(0, idx0, isem0)
  idx_load(1, idx1, isem1)

  def chunk_body(h, carry):
    for par in (0, 1):
      idx_v, sub_v, g_v, isem, gsem, wsem = bufs[par]
      idx_p, sub_p, g_p, isem_p, gsem_p, wsem_p = bufs[1 - par]
      g = h * 2 + par

      @pl.when(g >= 2)
      def _():
        write_wait(g - 2, g_v, wsem)

      idx_wait(g, idx_v, isem)
      offset_add(idx_v, sub_v)
      gather_fire(idx_v, g_v, gsem)

      @pl.when(g >= 1)
      def _():
        gather_wait(idx_p, g_p, gsem_p)

        @pl.when(g + 1 < N_CHUNKS)
        def _():
          idx_load(g + 1, idx_p, isem_p)

        extract(g_p, sub_p)
        write_fire(g - 1, g_p, wsem_p)
    return carry

  lax.fori_loop(0, N_CHUNKS // 2, chunk_body, 0)

  lastp = (N_CHUNKS - 1) & 1
  idx_l, sub_l, g_l, isem_l, gsem_l, wsem_l = bufs[lastp]
  gather_wait(idx_l, g_l, gsem_l)
  extract(g_l, sub_l)
  write_fire(N_CHUNKS - 1, g_l, wsem_l)
  write_wait(N_CHUNKS - 2, g0 if lastp else g1,
             wsem0 if lastp else wsem1)
  write_wait(N_CHUNKS - 1, g_l, wsem_l)


@jax.jit
def kernel(features, tables):
  idx = features.reshape(N_FEATURES, N).T.reshape(N * N_FEATURES)
  offs = jnp.tile(jnp.arange(N_FEATURES, dtype=jnp.int32) * VOCAB, B)
  flat = tables.reshape(N_FEATURES * VOCAB * EMBED)
  flat = jnp.concatenate([flat, jnp.zeros(64, jnp.float32)])
  table = flat.reshape(GROUPS, 128)
  mesh = plsc.VectorSubcoreMesh(core_axis_name="c", subcore_axis_name="s")
  out = pl.kernel(
      _embed_body,
      out_type=jax.ShapeDtypeStruct((N * N_FEATURES // 4, 128), jnp.float32),
      mesh=mesh,
      scratch_types=[
          pltpu.VMEM((ROWS,), jnp.int32),
          pltpu.VMEM((ROWS,), jnp.int32),
          pltpu.VMEM((ROWS,), jnp.int32),
          pltpu.VMEM((ROWS,), jnp.int32),
          pltpu.VMEM((ROWS,), jnp.int32),
          pltpu.VMEM((ROWS, 128), jnp.float32),
          pltpu.VMEM((ROWS, 128), jnp.float32),
          pltpu.SemaphoreType.DMA,
          pltpu.SemaphoreType.DMA,
          pltpu.SemaphoreType.DMA,
          pltpu.SemaphoreType.DMA,
          pltpu.SemaphoreType.DMA,
          pltpu.SemaphoreType.DMA,
      ],
      compiler_params=pltpu.CompilerParams(needs_layout_passes=False),
  )(idx, offs, table)
  return out.reshape(BATCH, SEQ, N_FEATURES * EMBED)
